# baseline (speedup 1.0000x reference)
import jax
import jax.numpy as jnp
from jax.experimental import pallas as pl
from jax.experimental.pallas import tpu as pltpu

EMBED = 768
NEXP = 8
NTOK = 2048
DFF = 3072
BD = 512
NJ = DFF // BD
NSTEP = NEXP * NJ
RSPLIT = 4
RB = NTOK // RSPLIT
NQ = 4
QR = EMBED // NQ


def _fused_body(x_ref, wr_ref, wg_hbm, wu_hbm, wd_hbm, out_ref, lb_ref,
                wbuf, acc_ref, xbf_ref, xw_ref, w8_ref, wdbuf,
                sems, wdsem):
    def wcopies(k, s):
        col = pl.ds(k * BD, BD)
        cps = []
        for q in range(NQ):
            qr = pl.ds(q * QR, QR)
            cps.append(pltpu.make_async_copy(
                wg_hbm.at[qr, col], wbuf.at[s, 0, qr], sems.at[s]))
            cps.append(pltpu.make_async_copy(
                wu_hbm.at[qr, col], wbuf.at[s, 1, qr], sems.at[s]))
        return cps

    def wdcopy(j):
        return pltpu.make_async_copy(wd_hbm.at[pl.ds(j * BD, BD), :],
                                     wdbuf.at[j % 2], wdsem)

    for cp in wcopies(0, 0):
        cp.start()
    wdcopy(0).start()
    wdcopy(1).start()

    x = x_ref[...]
    logits = jnp.dot(x, wr_ref[...], preferred_element_type=jnp.float32)
    m = jnp.max(logits, axis=1, keepdims=True)
    el = jnp.exp(logits - m)
    p = el / jnp.sum(el, axis=1, keepdims=True)
    idx8 = jax.lax.broadcasted_iota(jnp.int32, (NTOK, NEXP), 1)
    m1 = jnp.max(p, axis=1, keepdims=True)
    i1 = jnp.min(jnp.where(p == m1, idx8, NEXP), axis=1, keepdims=True)
    sel1 = idx8 == i1
    p2 = jnp.where(sel1, -1.0, p)
    m2 = jnp.max(p2, axis=1, keepdims=True)
    i2 = jnp.min(jnp.where(p2 == m2, idx8, NEXP), axis=1, keepdims=True)
    sel2 = idx8 == i2
    ssum = m1 + m2 + 1e-10
    w8_ref[...] = jnp.where(sel1, m1 / ssum,
                            jnp.where(sel2, m2 / ssum, 0.0))
    ep = jnp.mean(p, axis=0, keepdims=True)
    lb = NEXP * jnp.sum(ep * jnp.log(ep * NEXP + 1e-10))
    lb_ref[...] = jnp.reshape(lb, (1, 1))
    xbf_ref[...] = x.astype(jnp.bfloat16)

    for cp in wcopies(1, 1):
        cp.start()

    def step(k, carry):
        s = k % 3
        e = k // NJ
        j = k - e * NJ

        @pl.when(k + 2 < NSTEP)
        def _():
            for cp in wcopies(k + 2, (k + 2) % 3):
                cp.start()

        for cp in wcopies(k, s):
            cp.wait()

        @pl.when(j == 0)
        def _():
            ohe = (jax.lax.broadcasted_iota(jnp.int32, (1, NEXP), 1) == e
                   ).astype(jnp.float32)
            wcol = jnp.sum(w8_ref[...] * ohe, axis=1, keepdims=True)
            xw_ref[...] = (xbf_ref[...].astype(jnp.float32) * wcol
                           ).astype(jnp.bfloat16)

        @pl.when(e == NEXP - 1)
        def _():
            wdcopy(j).wait()

        dcol = pl.ds(j * BD, BD)
        wgb = wbuf[s, 0].astype(jnp.bfloat16)
        wub = wbuf[s, 1].astype(jnp.bfloat16)
        for r in range(RSPLIT):
            rows = pl.ds(r * RB, RB)
            g = jnp.dot(xbf_ref[rows, :], wgb,
                        preferred_element_type=jnp.float32)
            u = jnp.dot(xw_ref[rows, :], wub,
                        preferred_element_type=jnp.float32)
            h = (g * jax.lax.logistic(g) * u).astype(jnp.bfloat16)

            @pl.when(e == 0)
            def _():
                acc_ref[rows, dcol] = h

            @pl.when(jnp.logical_and(e != 0, e != NEXP - 1))
            def _():
                acc_ref[rows, dcol] += h

            @pl.when(e == NEXP - 1)
            def _():
                hfin = acc_ref[rows, dcol] + h
                y = jnp.dot(hfin, wdbuf[j % 2].astype(jnp.bfloat16),
                            preferred_element_type=jnp.float32)

                @pl.when(j == 0)
                def _():
                    out_ref[rows, :] = y

                @pl.when(j != 0)
                def _():
                    out_ref[rows, :] += y

        if True:
            @pl.when(jnp.logical_and(e == NEXP - 1, j + 2 < NJ))
            def _():
                wdcopy(j + 2).start()

        return carry

    jax.lax.fori_loop(0, NSTEP, step, 0)


def _fused_call(x_flat, W_router, W_gate, W_up, W_down):
    return pl.pallas_call(
        _fused_body,
        in_specs=[
            pl.BlockSpec(memory_space=pltpu.VMEM),
            pl.BlockSpec(memory_space=pltpu.VMEM),
            pl.BlockSpec(memory_space=pl.ANY),
            pl.BlockSpec(memory_space=pl.ANY),
            pl.BlockSpec(memory_space=pl.ANY),
        ],
        out_specs=(
            pl.BlockSpec(memory_space=pltpu.VMEM),
            pl.BlockSpec(memory_space=pltpu.VMEM),
        ),
        out_shape=(
            jax.ShapeDtypeStruct((NTOK, EMBED), jnp.float32),
            jax.ShapeDtypeStruct((1, 1), jnp.float32),
        ),
        scratch_shapes=[
            pltpu.VMEM((3, 2, EMBED, BD), jnp.float32),
            pltpu.VMEM((NTOK, DFF), jnp.bfloat16),
            pltpu.VMEM((NTOK, EMBED), jnp.bfloat16),
            pltpu.VMEM((NTOK, EMBED), jnp.bfloat16),
            pltpu.VMEM((NTOK, NEXP), jnp.float32),
            pltpu.VMEM((2, BD, EMBED), jnp.float32),
            pltpu.SemaphoreType.DMA((3,)),
            pltpu.SemaphoreType.DMA,
        ],
    )(x_flat, W_router, W_gate, W_up, W_down)


def kernel(x, W_router, W_gate, W_up, W_down):
    x_flat = x.reshape(NTOK, EMBED)
    out, lb = _fused_call(x_flat, W_router, W_gate, W_up, W_down)
    return out.reshape(x.shape), lb[0, 0]


# speedup vs baseline: 1.0985x; 1.0985x over previous
import jax
import jax.numpy as jnp
from jax.experimental import pallas as pl
from jax.experimental.pallas import tpu as pltpu

EMBED = 768
NEXP = 8
NTOK = 2048
DFF = 3072
BD = 768
NJ = DFF // BD
NSTEP = NEXP * NJ
RSPLIT = 4
RB = NTOK // RSPLIT
NQ = 4
QR = EMBED // NQ


def _fused_body(x_ref, wr_ref, wg_hbm, wu_hbm, wd_hbm, out_ref, lb_ref,
                wbuf, acc_ref, xbf_ref, xw_ref, w8_ref, wdbuf,
                sems, wdsem):
    def wcopies(k, s):
        col = pl.ds(k * BD, BD)
        cps = []
        for q in range(NQ):
            qr = pl.ds(q * QR, QR)
            cps.append(pltpu.make_async_copy(
                wg_hbm.at[qr, col], wbuf.at[s, 0, qr], sems.at[s]))
        for q in range(NQ):
            qr = pl.ds(q * QR, QR)
            cps.append(pltpu.make_async_copy(
                wu_hbm.at[qr, col], wbuf.at[s, 1, qr], sems.at[s]))
        return cps

    def wdcopy(j):
        return pltpu.make_async_copy(wd_hbm.at[pl.ds(j * BD, BD), :],
                                     wdbuf.at[j % 2], wdsem)

    for cp in wcopies(0, 0):
        cp.start()
    wdcopy(0).start()
    wdcopy(1).start()

    x = x_ref[...]
    logits = jnp.dot(x, wr_ref[...], preferred_element_type=jnp.float32)
    m = jnp.max(logits, axis=1, keepdims=True)
    el = jnp.exp(logits - m)
    p = el / jnp.sum(el, axis=1, keepdims=True)
    idx8 = jax.lax.broadcasted_iota(jnp.int32, (NTOK, NEXP), 1)
    m1 = jnp.max(p, axis=1, keepdims=True)
    i1 = jnp.min(jnp.where(p == m1, idx8, NEXP), axis=1, keepdims=True)
    sel1 = idx8 == i1
    p2 = jnp.where(sel1, -1.0, p)
    m2 = jnp.max(p2, axis=1, keepdims=True)
    i2 = jnp.min(jnp.where(p2 == m2, idx8, NEXP), axis=1, keepdims=True)
    sel2 = idx8 == i2
    ssum = m1 + m2 + 1e-10
    w8_ref[...] = jnp.where(sel1, m1 / ssum,
                            jnp.where(sel2, m2 / ssum, 0.0))
    ep = jnp.mean(p, axis=0, keepdims=True)
    lb = NEXP * jnp.sum(ep * jnp.log(ep * NEXP + 1e-10))
    lb_ref[...] = jnp.reshape(lb, (1, 1))
    xbf_ref[...] = x.astype(jnp.bfloat16)

    def step(k, carry):
        s = k % 2
        e = k // NJ
        j = k - e * NJ

        @pl.when(k + 1 < NSTEP)
        def _():
            for cp in wcopies(k + 1, (k + 1) % 2)[:NQ]:
                cp.start()

        for cp in wcopies(k, s):
            cp.wait()

        @pl.when(j == 0)
        def _():
            ohe = (jax.lax.broadcasted_iota(jnp.int32, (1, NEXP), 1) == e
                   ).astype(jnp.float32)
            wcol = jnp.sum(w8_ref[...] * ohe, axis=1, keepdims=True)
            xw_ref[...] = (xbf_ref[...].astype(jnp.float32) * wcol
                           ).astype(jnp.bfloat16)

        @pl.when(e == NEXP - 1)
        def _():
            wdcopy(j).wait()

        dcol = pl.ds(j * BD, BD)
        wgb = wbuf[s, 0].astype(jnp.bfloat16)

        @pl.when(k + 1 < NSTEP)
        def _():
            for cp in wcopies(k + 1, (k + 1) % 2)[NQ:]:
                cp.start()

        wub = wbuf[s, 1].astype(jnp.bfloat16)
        for r in range(RSPLIT):
            rows = pl.ds(r * RB, RB)
            g = jnp.dot(xbf_ref[rows, :], wgb,
                        preferred_element_type=jnp.float32)
            u = jnp.dot(xw_ref[rows, :], wub,
                        preferred_element_type=jnp.float32)
            h = (g * jax.lax.logistic(g) * u).astype(jnp.bfloat16)

            @pl.when(e == 0)
            def _():
                acc_ref[rows, dcol] = h

            @pl.when(jnp.logical_and(e != 0, e != NEXP - 1))
            def _():
                acc_ref[rows, dcol] += h

            @pl.when(e == NEXP - 1)
            def _():
                hfin = acc_ref[rows, dcol] + h
                y = jnp.dot(hfin, wdbuf[j % 2].astype(jnp.bfloat16),
                            preferred_element_type=jnp.float32)

                @pl.when(j == 0)
                def _():
                    out_ref[rows, :] = y

                @pl.when(j != 0)
                def _():
                    out_ref[rows, :] += y

        if True:
            @pl.when(jnp.logical_and(e == NEXP - 1, j + 2 < NJ))
            def _():
                wdcopy(j + 2).start()

        return carry

    jax.lax.fori_loop(0, NSTEP, step, 0)


def _fused_call(x_flat, W_router, W_gate, W_up, W_down):
    return pl.pallas_call(
        _fused_body,
        in_specs=[
            pl.BlockSpec(memory_space=pltpu.VMEM),
            pl.BlockSpec(memory_space=pltpu.VMEM),
            pl.BlockSpec(memory_space=pl.ANY),
            pl.BlockSpec(memory_space=pl.ANY),
            pl.BlockSpec(memory_space=pl.ANY),
        ],
        out_specs=(
            pl.BlockSpec(memory_space=pltpu.VMEM),
            pl.BlockSpec(memory_space=pltpu.VMEM),
        ),
        out_shape=(
            jax.ShapeDtypeStruct((NTOK, EMBED), jnp.float32),
            jax.ShapeDtypeStruct((1, 1), jnp.float32),
        ),
        scratch_shapes=[
            pltpu.VMEM((2, 2, EMBED, BD), jnp.float32),
            pltpu.VMEM((NTOK, DFF), jnp.bfloat16),
            pltpu.VMEM((NTOK, EMBED), jnp.bfloat16),
            pltpu.VMEM((NTOK, EMBED), jnp.bfloat16),
            pltpu.VMEM((NTOK, NEXP), jnp.float32),
            pltpu.VMEM((2, BD, EMBED), jnp.float32),
            pltpu.SemaphoreType.DMA((2,)),
            pltpu.SemaphoreType.DMA,
        ],
    )(x_flat, W_router, W_gate, W_up, W_down)


def kernel(x, W_router, W_gate, W_up, W_down):
    x_flat = x.reshape(NTOK, EMBED)
    out, lb = _fused_call(x_flat, W_router, W_gate, W_up, W_down)
    return out.reshape(x.shape), lb[0, 0]


# speedup vs baseline: 1.1045x; 1.0055x over previous
import jax
import jax.numpy as jnp
from jax.experimental import pallas as pl
from jax.experimental.pallas import tpu as pltpu

EMBED = 768
NEXP = 8
NTOK = 2048
DFF = 3072
BD = 768
NJ = DFF // BD
NSTEP = NEXP * NJ
RSPLIT = 4
RB = NTOK // RSPLIT
NQ = 4
QR = EMBED // NQ


def _fused_body(x_ref, wr_ref, wg_hbm, wu_hbm, wd_hbm, out_ref, lb_ref,
                wbuf, acc_ref, xbf_ref, xw_ref, w8_ref, wdbuf,
                sems, wdsem):
    def wcopies(k, s):
        col = pl.ds(k * BD, BD)
        cps = []
        for q in range(NQ):
            qr = pl.ds(q * QR, QR)
            cps.append(pltpu.make_async_copy(
                wg_hbm.at[qr, col], wbuf.at[s, 0, qr], sems.at[s]))
            cps.append(pltpu.make_async_copy(
                wu_hbm.at[qr, col], wbuf.at[s, 1, qr], sems.at[s]))
        return cps

    def wdcopy(j):
        return pltpu.make_async_copy(wd_hbm.at[pl.ds(j * BD, BD), :],
                                     wdbuf.at[j % 2], wdsem)

    for cp in wcopies(0, 0):
        cp.start()
    wdcopy(0).start()
    wdcopy(1).start()

    x = x_ref[...]
    logits = jnp.dot(x, wr_ref[...], preferred_element_type=jnp.float32)
    m = jnp.max(logits, axis=1, keepdims=True)
    el = jnp.exp(logits - m)
    p = el / jnp.sum(el, axis=1, keepdims=True)
    idx8 = jax.lax.broadcasted_iota(jnp.int32, (NTOK, NEXP), 1)
    m1 = jnp.max(p, axis=1, keepdims=True)
    i1 = jnp.min(jnp.where(p == m1, idx8, NEXP), axis=1, keepdims=True)
    sel1 = idx8 == i1
    p2 = jnp.where(sel1, -1.0, p)
    m2 = jnp.max(p2, axis=1, keepdims=True)
    i2 = jnp.min(jnp.where(p2 == m2, idx8, NEXP), axis=1, keepdims=True)
    sel2 = idx8 == i2
    ssum = m1 + m2 + 1e-10
    w8_ref[...] = jnp.where(sel1, m1 / ssum,
                            jnp.where(sel2, m2 / ssum, 0.0))
    ep = jnp.mean(p, axis=0, keepdims=True)
    lb = NEXP * jnp.sum(ep * jnp.log(ep * NEXP + 1e-10))
    lb_ref[...] = jnp.reshape(lb, (1, 1))
    xbf_ref[...] = x.astype(jnp.bfloat16)

    def step(k, carry):
        s = k % 2
        e = k // NJ
        j = k - e * NJ

        @pl.when(k + 1 < NSTEP)
        def _():
            for cp in wcopies(k + 1, (k + 1) % 2):
                cp.start()

        for cp in wcopies(k, s):
            cp.wait()

        @pl.when(j == 0)
        def _():
            ohe = (jax.lax.broadcasted_iota(jnp.int32, (1, NEXP), 1) == e
                   ).astype(jnp.float32)
            wcol = jnp.sum(w8_ref[...] * ohe, axis=1, keepdims=True)
            xw_ref[...] = (xbf_ref[...].astype(jnp.float32) * wcol
                           ).astype(jnp.bfloat16)

        @pl.when(e == NEXP - 1)
        def _():
            wdcopy(j).wait()

        dcol = pl.ds(j * BD, BD)
        wgb = wbuf[s, 0].astype(jnp.bfloat16)
        wub = wbuf[s, 1].astype(jnp.bfloat16)
        for r in range(RSPLIT):
            rows = pl.ds(r * RB, RB)
            g = jnp.dot(xbf_ref[rows, :], wgb,
                        preferred_element_type=jnp.float32)
            u = jnp.dot(xw_ref[rows, :], wub,
                        preferred_element_type=jnp.float32)
            h = (g * jax.lax.logistic(g) * u).astype(jnp.bfloat16)

            @pl.when(e == 0)
            def _():
                acc_ref[rows, dcol] = h

            @pl.when(jnp.logical_and(e != 0, e != NEXP - 1))
            def _():
                acc_ref[rows, dcol] += h

            @pl.when(e == NEXP - 1)
            def _():
                hfin = acc_ref[rows, dcol] + h
                y = jnp.dot(hfin, wdbuf[j % 2].astype(jnp.bfloat16),
                            preferred_element_type=jnp.float32)

                @pl.when(j == 0)
                def _():
                    out_ref[rows, :] = y

                @pl.when(j != 0)
                def _():
                    out_ref[rows, :] += y

        if True:
            @pl.when(jnp.logical_and(e == NEXP - 1, j + 2 < NJ))
            def _():
                wdcopy(j + 2).start()

        return carry

    jax.lax.fori_loop(0, NSTEP, step, 0)


def _fused_call(x_flat, W_router, W_gate, W_up, W_down):
    return pl.pallas_call(
        _fused_body,
        in_specs=[
            pl.BlockSpec(memory_space=pltpu.VMEM),
            pl.BlockSpec(memory_space=pltpu.VMEM),
            pl.BlockSpec(memory_space=pl.ANY),
            pl.BlockSpec(memory_space=pl.ANY),
            pl.BlockSpec(memory_space=pl.ANY),
        ],
        out_specs=(
            pl.BlockSpec(memory_space=pltpu.VMEM),
            pl.BlockSpec(memory_space=pltpu.VMEM),
        ),
        out_shape=(
            jax.ShapeDtypeStruct((NTOK, EMBED), jnp.float32),
            jax.ShapeDtypeStruct((1, 1), jnp.float32),
        ),
        scratch_shapes=[
            pltpu.VMEM((2, 2, EMBED, BD), jnp.float32),
            pltpu.VMEM((NTOK, DFF), jnp.bfloat16),
            pltpu.VMEM((NTOK, EMBED), jnp.bfloat16),
            pltpu.VMEM((NTOK, EMBED), jnp.bfloat16),
            pltpu.VMEM((NTOK, NEXP), jnp.float32),
            pltpu.VMEM((2, BD, EMBED), jnp.float32),
            pltpu.SemaphoreType.DMA((2,)),
            pltpu.SemaphoreType.DMA,
        ],
    )(x_flat, W_router, W_gate, W_up, W_down)


def kernel(x, W_router, W_gate, W_up, W_down):
    x_flat = x.reshape(NTOK, EMBED)
    out, lb = _fused_call(x_flat, W_router, W_gate, W_up, W_down)
    return out.reshape(x.shape), lb[0, 0]
